# baseline (speedup 1.0000x reference)
import functools

import jax
import jax.numpy as jnp
from jax import lax
from jax.experimental import pallas as pl
from jax.experimental.pallas import tpu as pltpu
from jax.experimental.pallas import tpu_sc as plsc

BATCH = 4096
SEQ = 50
EMBED = 64
LANES = 16
NGROUP = EMBED // LANES

NC, NS = 2, 16
NW = NC * NS
ROWS_W = BATCH // NW
CB = 2
CHUNK_TOK = CB * SEQ
NCHUNK = ROWS_W // CB
TOK_W = ROWS_W * SEQ


def _body(idx_hbm, mask_hbm, table_hbm, out_hbm, idx_v, mask_v, rows_v,
          out_v, sem):
    wid = lax.axis_index("s") * NC + lax.axis_index("c")

    pltpu.sync_copy(idx_hbm.at[pl.ds(wid * NCHUNK, NCHUNK), :], idx_v)
    pltpu.sync_copy(mask_hbm.at[pl.ds(wid * TOK_W, TOK_W)],
                    mask_v.at[pl.ds(0, TOK_W)])

    def chunk(g, carry):
        pltpu.async_copy(table_hbm.at[idx_v.at[g]], rows_v, sem).wait()

        zero = jnp.zeros((LANES,), jnp.float32)

        def s_step(s, acc):
            new = []
            for j in range(CB):
                a = acc[j]
                tok = g * CHUNK_TOK + j * SEQ + s
                m = mask_v[pl.ds(tok, LANES)][0]
                r = j * SEQ + s
                vals = [a[k] + rows_v[r, pl.ds(k * LANES, LANES)] * m
                        for k in range(NGROUP)]
                vals.append(a[NGROUP] + m)
                new.append(tuple(vals))
            return tuple(new)

        init = tuple(tuple(zero for _ in range(NGROUP)) + (jnp.float32(0.0),)
                     for _ in range(CB))
        acc = lax.fori_loop(0, SEQ, s_step, init)

        for j in range(CB):
            denom = jnp.broadcast_to(jnp.maximum(acc[j][NGROUP], 1.0),
                                     (LANES,))
            for k in range(NGROUP):
                out_v[g * CB + j, pl.ds(k * LANES, LANES)] = acc[j][k] / denom
        return carry

    lax.fori_loop(0, NCHUNK, chunk, 0)

    pltpu.sync_copy(out_v, out_hbm.at[pl.ds(wid * ROWS_W, ROWS_W), :])


@jax.jit
def _embed(idx2, maskf, table):
    mesh = plsc.VectorSubcoreMesh(core_axis_name="c", subcore_axis_name="s")
    f = pl.kernel(
        _body,
        out_type=jax.ShapeDtypeStruct((BATCH, EMBED), jnp.float32),
        mesh=mesh,
        scratch_types=[
            pltpu.VMEM((NCHUNK, CHUNK_TOK), jnp.int32),
            pltpu.VMEM((TOK_W + LANES,), jnp.float32),
            pltpu.VMEM((CHUNK_TOK, EMBED), jnp.float32),
            pltpu.VMEM((ROWS_W, EMBED), jnp.float32),
            pltpu.SemaphoreType.DMA,
        ],
        compiler_params=pltpu.CompilerParams(use_tc_tiling_on_sc=False),
    )
    return f(idx2, maskf, table)


def kernel(token_indices, mask, embedding_table):
    idx2 = token_indices.reshape(BATCH // CB, CHUNK_TOK)
    maskf = mask.reshape(-1)
    return _embed(idx2, maskf, embedding_table)


# speedup vs baseline: 1.0509x; 1.0509x over previous
import jax
import jax.numpy as jnp
from jax import lax
from jax.experimental import pallas as pl
from jax.experimental.pallas import tpu as pltpu
from jax.experimental.pallas import tpu_sc as plsc

BATCH = 4096
SEQ = 50
EMBED = 64
LANES = 16
NGROUP = EMBED // LANES

NC, NS = 2, 16
NW = NC * NS
ROWS_W = BATCH // NW
CB = 2
CHUNK_TOK = CB * SEQ
NCHUNK = ROWS_W // CB
TOK_W = ROWS_W * SEQ
NMV = (SEQ + LANES - 1) // LANES + 1


def _body(idx_hbm, mask_hbm, table_hbm, out_hbm, idx_v, mask_v, rows0, rows1,
          out_v, sem0, sem1):
    wid = lax.axis_index("s") * NC + lax.axis_index("c")
    rows_b = (rows0, rows1)
    sems = (sem0, sem1)

    pltpu.sync_copy(idx_hbm.at[pl.ds(wid * NCHUNK, NCHUNK), :], idx_v)
    pltpu.sync_copy(mask_hbm.at[pl.ds(wid * TOK_W, TOK_W)],
                    mask_v.at[pl.ds(0, TOK_W)])

    lane_ids = [jnp.full((LANES,), i, jnp.int32) for i in range(LANES)]
    tail_keep = (lax.iota(jnp.int32, LANES) < (SEQ - 3 * LANES)).astype(
        jnp.float32)

    def start(g, b):
        pltpu.make_async_copy(
            table_hbm.at[idx_v.at[g]], rows_b[b], sems[b]).start()

    def compute(g, b):
        rows = rows_b[b]
        zero = jnp.zeros((LANES,), jnp.float32)

        def s_step(s, acc_all):
            new = []
            for j in range(CB):
                a = acc_all[j]
                tok = g * CHUNK_TOK + j * SEQ + s
                m = mask_v[pl.ds(tok, LANES)][0]
                r = j * SEQ + s
                vals = [a[k] + rows[r, pl.ds(k * LANES, LANES)] * m
                        for k in range(NGROUP)]
                vals.append(a[NGROUP] + m)
                new.append(tuple(vals))
            return tuple(new)

        init = tuple(tuple(zero for _ in range(NGROUP)) + (jnp.float32(0.0),)
                     for _ in range(CB))
        acc_all = lax.fori_loop(0, SEQ, s_step, init)
        for j in range(CB):
            denom = jnp.broadcast_to(
                jnp.maximum(acc_all[j][NGROUP], 1.0), (LANES,))
            for k in range(NGROUP):
                out_v[g * CB + j, pl.ds(k * LANES, LANES)] = (
                    acc_all[j][k] / denom)

    start(0, 0)
    start(1, 1)

    def tb(t, carry):
        for b in range(2):
            g = 2 * t + b
            pltpu.make_async_copy(
                table_hbm.at[idx_v.at[g]], rows_b[b], sems[b]).wait()
            compute(g, b)

            @pl.when(g + 2 < NCHUNK)
            def _():
                start(g + 2, b)
        return carry

    lax.fori_loop(0, NCHUNK // 2, tb, 0)

    pltpu.sync_copy(out_v, out_hbm.at[pl.ds(wid * ROWS_W, ROWS_W), :])


@jax.jit
def _embed(idx2, maskf, table):
    mesh = plsc.VectorSubcoreMesh(core_axis_name="c", subcore_axis_name="s")
    f = pl.kernel(
        _body,
        out_type=jax.ShapeDtypeStruct((BATCH, EMBED), jnp.float32),
        mesh=mesh,
        scratch_types=[
            pltpu.VMEM((NCHUNK, CHUNK_TOK), jnp.int32),
            pltpu.VMEM((TOK_W + LANES,), jnp.float32),
            pltpu.VMEM((CHUNK_TOK, EMBED), jnp.float32),
            pltpu.VMEM((CHUNK_TOK, EMBED), jnp.float32),
            pltpu.VMEM((ROWS_W, EMBED), jnp.float32),
            pltpu.SemaphoreType.DMA,
            pltpu.SemaphoreType.DMA,
        ],
        compiler_params=pltpu.CompilerParams(use_tc_tiling_on_sc=False),
    )
    return f(idx2, maskf, table)


def kernel(token_indices, mask, embedding_table):
    idx2 = token_indices.reshape(BATCH // CB, CHUNK_TOK)
    maskf = mask.reshape(-1)
    return _embed(idx2, maskf, embedding_table)


# speedup vs baseline: 1.0791x; 1.0268x over previous
import jax
import jax.numpy as jnp
from jax import lax
from jax.experimental import pallas as pl
from jax.experimental.pallas import tpu as pltpu
from jax.experimental.pallas import tpu_sc as plsc

BATCH = 4096
SEQ = 50
EMBED = 64
LANES = 16
NGROUP = EMBED // LANES

NC, NS = 2, 16
NW = NC * NS
ROWS_W = BATCH // NW
CB = 2
CHUNK_TOK = CB * SEQ
NCHUNK = ROWS_W // CB
TOK_W = ROWS_W * SEQ
NBUF = 4


def _body(idx_hbm, mask_hbm, table_hbm, out_hbm, idx_v, mask_v, rows0, rows1,
          rows2, rows3, out_v, sem0, sem1, sem2, sem3):
    wid = lax.axis_index("s") * NC + lax.axis_index("c")
    rows_b = (rows0, rows1, rows2, rows3)
    sems = (sem0, sem1, sem2, sem3)

    pltpu.sync_copy(idx_hbm.at[pl.ds(wid * NCHUNK, NCHUNK), :], idx_v)
    pltpu.sync_copy(mask_hbm.at[pl.ds(wid * TOK_W, TOK_W)],
                    mask_v.at[pl.ds(0, TOK_W)])

    def start(g, b):
        pltpu.make_async_copy(
            table_hbm.at[idx_v.at[g]], rows_b[b], sems[b]).start()

    def compute(g, b):
        rows = rows_b[b]
        zero = jnp.zeros((LANES,), jnp.float32)

        def s_step(s, acc_all):
            new = []
            for j in range(CB):
                a = acc_all[j]
                tok = g * CHUNK_TOK + j * SEQ + s
                m = mask_v[pl.ds(tok, LANES)][0]
                r = j * SEQ + s
                vals = [a[k] + rows[r, pl.ds(k * LANES, LANES)] * m
                        for k in range(NGROUP)]
                vals.append(a[NGROUP] + m)
                new.append(tuple(vals))
            return tuple(new)

        init = tuple(tuple(zero for _ in range(NGROUP)) + (jnp.float32(0.0),)
                     for _ in range(CB))
        acc_all = lax.fori_loop(0, SEQ, s_step, init)
        for j in range(CB):
            denom = jnp.broadcast_to(
                jnp.maximum(acc_all[j][NGROUP], 1.0), (LANES,))
            for k in range(NGROUP):
                out_v[g * CB + j, pl.ds(k * LANES, LANES)] = (
                    acc_all[j][k] / denom)

    for b in range(NBUF):
        start(b, b)

    def tb(t, carry):
        for b in range(NBUF):
            g = NBUF * t + b
            pltpu.make_async_copy(
                table_hbm.at[idx_v.at[g]], rows_b[b], sems[b]).wait()
            compute(g, b)

            @pl.when(g + NBUF < NCHUNK)
            def _():
                start(g + NBUF, b)
        return carry

    lax.fori_loop(0, NCHUNK // NBUF, tb, 0)

    pltpu.sync_copy(out_v, out_hbm.at[pl.ds(wid * ROWS_W, ROWS_W), :])


@jax.jit
def _embed(idx2, maskf, table):
    mesh = plsc.VectorSubcoreMesh(core_axis_name="c", subcore_axis_name="s")
    f = pl.kernel(
        _body,
        out_type=jax.ShapeDtypeStruct((BATCH, EMBED), jnp.float32),
        mesh=mesh,
        scratch_types=[
            pltpu.VMEM((NCHUNK, CHUNK_TOK), jnp.int32),
            pltpu.VMEM((TOK_W + LANES,), jnp.float32),
            pltpu.VMEM((CHUNK_TOK, EMBED), jnp.float32),
            pltpu.VMEM((CHUNK_TOK, EMBED), jnp.float32),
            pltpu.VMEM((CHUNK_TOK, EMBED), jnp.float32),
            pltpu.VMEM((CHUNK_TOK, EMBED), jnp.float32),
            pltpu.VMEM((ROWS_W, EMBED), jnp.float32),
            pltpu.SemaphoreType.DMA,
            pltpu.SemaphoreType.DMA,
            pltpu.SemaphoreType.DMA,
            pltpu.SemaphoreType.DMA,
        ],
        compiler_params=pltpu.CompilerParams(use_tc_tiling_on_sc=False),
    )
    return f(idx2, maskf, table)


def kernel(token_indices, mask, embedding_table):
    idx2 = token_indices.reshape(BATCH // CB, CHUNK_TOK)
    maskf = mask.reshape(-1)
    return _embed(idx2, maskf, embedding_table)
